# baseline (speedup 1.0000x reference)
import functools

import jax
import jax.numpy as jnp
from jax import lax
from jax.experimental import pallas as pl
from jax.experimental.pallas import tpu as pltpu
from jax.experimental.pallas import tpu_sc as plsc

_B, _C, _H, _W = 2, 96, 512, 512
_PLANES = _B * _C
_PLANE = _H * _W
_NC, _NS, _L = 2, 16, 16
_NW = _NC * _NS
_PPW_SC = 3
_SC_PLANES = _NW * _PPW_SC
_TC_PLANES = _PLANES - _SC_PLANES
_ROWS = 64
_NCHUNK = _H // _ROWS
_ACC = 4
_STRIDE = _ACC * _L
_ITERS = _ROWS * _W // _STRIDE
_UNROLL = 4


def _sc_body(tgt, s0, s1, out, buf0, buf1, s0_v, s1_v, res_v, sem0, sem1):
    cid = lax.axis_index("c")
    sid = lax.axis_index("s")
    wid = sid * _NC + cid
    plane_base = wid * _PPW_SC

    bufs = (buf0, buf1)
    sems = (sem0, sem1)
    lanes = lax.iota(jnp.int32, _L)
    neg_inf = jnp.full((_L,), -jnp.inf, jnp.float32)
    zero_i = jnp.zeros((_L,), jnp.int32)

    pltpu.sync_copy(s0, s0_v)
    pltpu.sync_copy(s1, s1_v)
    ratio_h = lax.div(jnp.full((_L,), _H, jnp.int32), s0_v[...])
    ratio_w = lax.div(jnp.full((_L,), _W, jnp.int32), s1_v[...])

    def start_dma(c):
        j, k = divmod(c, _NCHUNK)
        src = tgt.at[plane_base + j, pl.ds(k * _ROWS, _ROWS), :]
        return pltpu.async_copy(src, bufs[c % 2], sems[c % 2])

    total = _PPW_SC * _NCHUNK
    descs = [None, None]
    descs[0] = start_dma(0)

    res = jnp.zeros((_L,), jnp.float32)

    for j in range(_PPW_SC):
        bv = [neg_inf] * _ACC
        bi = [zero_i] * _ACC
        for k in range(_NCHUNK):
            c = j * _NCHUNK + k
            if c + 1 < total:
                descs[(c + 1) % 2] = start_dma(c + 1)
            descs[c % 2].wait()
            buf = bufs[c % 2]

            def chunk_body(i, carry, buf=buf, k=k):
                v = list(carry[:_ACC])
                ii = list(carry[_ACC:])
                isplat = jnp.full((_L,), i, jnp.int32)
                f0 = (i - k * _ITERS) * _STRIDE
                for a in range(_ACC):
                    f = f0 + a * _L
                    r = lax.shift_right_logical(f, 9)
                    col = lax.bitwise_and(f, 511)
                    x = buf[r, pl.ds(col, _L)]
                    m = x > v[a]
                    v[a] = jnp.where(m, x, v[a])
                    ii[a] = jnp.where(m, isplat, ii[a])
                return tuple(v) + tuple(ii)

            carry = tuple(bv) + tuple(bi)
            carry = plsc.parallel_loop(
                k * _ITERS, (k + 1) * _ITERS, carry=carry,
                unroll=_UNROLL)(chunk_body)
            bv = list(carry[:_ACC])
            bi = list(carry[_ACC:])

        pv = None
        pi = None
        for a in range(_ACC):
            kv = lax.shift_right_logical(bi[a], 9)
            f = (bi[a] & 511) * _STRIDE + a * _L + lanes
            row = kv * _ROWS + lax.shift_right_logical(f, 9)
            col = lax.bitwise_and(f, 511)
            gi = row * _W + col
            if a == 0:
                pv, pi = bv[a], gi
            else:
                take = (bv[a] > pv) | ((bv[a] == pv) & (gi < pi))
                pv = jnp.where(take, bv[a], pv)
                pi = jnp.where(take, gi, pi)

        m = jnp.max(pv, axis=0)
        cand = jnp.where(pv == m, pi, jnp.int32(_PLANE))
        gidx = jnp.min(cand, axis=0)

        gv16 = jnp.full((_L,), gidx, jnp.int32)
        rowv = lax.div(gv16, jnp.full((_L,), _W, jnp.int32))
        colv = gv16 - rowv * _W
        rv = lax.div(rowv, ratio_h)
        cv = lax.div(colv, ratio_w)
        ok = (rv == 0) & (cv == 0)
        val = jnp.where(ok, jnp.float32(1.0), jnp.float32(0.0))
        res = jnp.where(lanes == j, val, res)

    res_v[...] = res
    pltpu.sync_copy(res_v, out.at[wid])


@functools.partial(
    pl.kernel,
    out_type=jax.ShapeDtypeStruct((_NW, _L), jnp.float32),
    mesh=plsc.VectorSubcoreMesh(core_axis_name="c", subcore_axis_name="s"),
    compiler_params=pltpu.CompilerParams(needs_layout_passes=False,
                                         use_tc_tiling_on_sc=True),
    scratch_types=[
        pltpu.VMEM((_ROWS, _W), jnp.float32),
        pltpu.VMEM((_ROWS, _W), jnp.float32),
        pltpu.VMEM((_L,), jnp.int32),
        pltpu.VMEM((_L,), jnp.int32),
        pltpu.VMEM((_L,), jnp.float32),
        pltpu.SemaphoreType.DMA,
        pltpu.SemaphoreType.DMA,
    ],
)
def _sc_argmax(tgt, s0, s1, out, buf0, buf1, s0_v, s1_v, res_v, sem0, sem1):
    _sc_body(tgt, s0, s1, out, buf0, buf1, s0_v, s1_v, res_v, sem0, sem1)


def _tc_body(params_ref, x_ref, o_ref):
    x = x_ref[...]
    m = jnp.max(x, axis=(1, 2), keepdims=True)
    iota = (lax.broadcasted_iota(jnp.int32, x.shape, 1) * _W +
            lax.broadcasted_iota(jnp.int32, x.shape, 2))
    idx = jnp.min(jnp.where(x == m, iota, jnp.int32(_PLANE)),
                  axis=(1, 2))
    rh = _H // params_ref[0, 0]
    rw = _W // params_ref[0, 1]
    row = idx // _W
    col = idx - row * _W
    ok = ((row // rh) == 0) & ((col // rw) == 0)
    val = jnp.where(ok, jnp.float32(1.0), jnp.float32(0.0))
    o_ref[...] = jnp.broadcast_to(val[:, None, None], (8, 1, 128))


def _tc_argmax(size2d, tgt3):
    out3 = pl.pallas_call(
        _tc_body,
        grid=(_TC_PLANES // 8,),
        in_specs=[
            pl.BlockSpec(memory_space=pltpu.SMEM),
            pl.BlockSpec((8, _H, _W), lambda g: (g + _SC_PLANES // 8, 0, 0)),
        ],
        out_specs=pl.BlockSpec((8, 1, 128), lambda g: (g, 0, 0)),
        out_shape=jax.ShapeDtypeStruct((_TC_PLANES, 1, 128), jnp.float32),
        compiler_params=pltpu.CompilerParams(
            vmem_limit_bytes=100 * 1024 * 1024),
    )(size2d, tgt3)
    return out3[:, 0, 0]


@jax.jit
def kernel(size, target):
    tgt3 = target.reshape(_PLANES, _H, _W)
    parts = []
    if _SC_PLANES:
        s0 = jnp.full((_L,), size[0], jnp.int32)
        s1 = jnp.full((_L,), size[1], jnp.int32)
        sc2d = _sc_argmax(tgt3, s0, s1)
        parts.append(sc2d[:, :_PPW_SC].reshape(_SC_PLANES))
    if _TC_PLANES:
        parts.append(
            _tc_argmax(size.reshape(1, 2), tgt3).reshape(_TC_PLANES))
    return jnp.concatenate(parts).reshape(_B, _C, 1, 1)


# speedup vs baseline: 1.3608x; 1.3608x over previous
import functools

import jax
import jax.numpy as jnp
from jax import lax
from jax.experimental import pallas as pl
from jax.experimental.pallas import tpu as pltpu
from jax.experimental.pallas import tpu_sc as plsc

_B, _C, _H, _W = 2, 96, 512, 512
_PLANES = _B * _C
_PLANE = _H * _W
_NC, _NS, _L = 2, 16, 16
_NW = _NC * _NS
_PPW_SC = 2
_SC_PLANES = _NW * _PPW_SC
_TC_PLANES = _PLANES - _SC_PLANES
_ROWS = 64
_NCHUNK = _H // _ROWS
_ACC = 4
_STRIDE = _ACC * _L
_ITERS = _ROWS * _W // _STRIDE
_UNROLL = 4


def _sc_body(tgt, s0, s1, out, buf0, buf1, s0_v, s1_v, res_v, sem0, sem1):
    cid = lax.axis_index("c")
    sid = lax.axis_index("s")
    wid = sid * _NC + cid
    plane_base = wid * _PPW_SC

    bufs = (buf0, buf1)
    sems = (sem0, sem1)
    lanes = lax.iota(jnp.int32, _L)
    neg_inf = jnp.full((_L,), -jnp.inf, jnp.float32)
    zero_i = jnp.zeros((_L,), jnp.int32)

    pltpu.sync_copy(s0, s0_v)
    pltpu.sync_copy(s1, s1_v)
    ratio_h = lax.div(jnp.full((_L,), _H, jnp.int32), s0_v[...])
    ratio_w = lax.div(jnp.full((_L,), _W, jnp.int32), s1_v[...])

    def start_dma(c):
        j, k = divmod(c, _NCHUNK)
        src = tgt.at[plane_base + j, pl.ds(k * _ROWS, _ROWS), :]
        return pltpu.async_copy(src, bufs[c % 2], sems[c % 2])

    total = _PPW_SC * _NCHUNK
    descs = [None, None]
    descs[0] = start_dma(0)

    res = jnp.zeros((_L,), jnp.float32)

    for j in range(_PPW_SC):
        bv = [neg_inf] * _ACC
        bi = [zero_i] * _ACC
        for k in range(_NCHUNK):
            c = j * _NCHUNK + k
            if c + 1 < total:
                descs[(c + 1) % 2] = start_dma(c + 1)
            descs[c % 2].wait()
            buf = bufs[c % 2]

            def chunk_body(i, carry, buf=buf, k=k):
                v = list(carry[:_ACC])
                ii = list(carry[_ACC:])
                isplat = jnp.full((_L,), i, jnp.int32)
                f0 = (i - k * _ITERS) * _STRIDE
                for a in range(_ACC):
                    f = f0 + a * _L
                    r = lax.shift_right_logical(f, 9)
                    col = lax.bitwise_and(f, 511)
                    x = buf[r, pl.ds(col, _L)]
                    m = x > v[a]
                    v[a] = jnp.where(m, x, v[a])
                    ii[a] = jnp.where(m, isplat, ii[a])
                return tuple(v) + tuple(ii)

            carry = tuple(bv) + tuple(bi)
            carry = plsc.parallel_loop(
                k * _ITERS, (k + 1) * _ITERS, carry=carry,
                unroll=_UNROLL)(chunk_body)
            bv = list(carry[:_ACC])
            bi = list(carry[_ACC:])

        pv = None
        pi = None
        for a in range(_ACC):
            kv = lax.shift_right_logical(bi[a], 9)
            f = (bi[a] & 511) * _STRIDE + a * _L + lanes
            row = kv * _ROWS + lax.shift_right_logical(f, 9)
            col = lax.bitwise_and(f, 511)
            gi = row * _W + col
            if a == 0:
                pv, pi = bv[a], gi
            else:
                take = (bv[a] > pv) | ((bv[a] == pv) & (gi < pi))
                pv = jnp.where(take, bv[a], pv)
                pi = jnp.where(take, gi, pi)

        m = jnp.max(pv, axis=0)
        cand = jnp.where(pv == m, pi, jnp.int32(_PLANE))
        gidx = jnp.min(cand, axis=0)

        gv16 = jnp.full((_L,), gidx, jnp.int32)
        rowv = lax.div(gv16, jnp.full((_L,), _W, jnp.int32))
        colv = gv16 - rowv * _W
        rv = lax.div(rowv, ratio_h)
        cv = lax.div(colv, ratio_w)
        ok = (rv == 0) & (cv == 0)
        val = jnp.where(ok, jnp.float32(1.0), jnp.float32(0.0))
        res = jnp.where(lanes == j, val, res)

    res_v[...] = res
    pltpu.sync_copy(res_v, out.at[wid])


@functools.partial(
    pl.kernel,
    out_type=jax.ShapeDtypeStruct((_NW, _L), jnp.float32),
    mesh=plsc.VectorSubcoreMesh(core_axis_name="c", subcore_axis_name="s"),
    compiler_params=pltpu.CompilerParams(needs_layout_passes=False,
                                         use_tc_tiling_on_sc=True),
    scratch_types=[
        pltpu.VMEM((_ROWS, _W), jnp.float32),
        pltpu.VMEM((_ROWS, _W), jnp.float32),
        pltpu.VMEM((_L,), jnp.int32),
        pltpu.VMEM((_L,), jnp.int32),
        pltpu.VMEM((_L,), jnp.float32),
        pltpu.SemaphoreType.DMA,
        pltpu.SemaphoreType.DMA,
    ],
)
def _sc_argmax(tgt, s0, s1, out, buf0, buf1, s0_v, s1_v, res_v, sem0, sem1):
    _sc_body(tgt, s0, s1, out, buf0, buf1, s0_v, s1_v, res_v, sem0, sem1)


_TC_BLK = 16


def _tc_body(params_ref, x_ref, o_ref):
    x = x_ref[...]
    m = jnp.max(x, axis=(1, 2), keepdims=True)
    iota = (lax.broadcasted_iota(jnp.int32, x.shape, 1) * _W +
            lax.broadcasted_iota(jnp.int32, x.shape, 2))
    idx = jnp.min(jnp.where(x == m, iota, jnp.int32(_PLANE)),
                  axis=(1, 2))
    rh = _H // params_ref[0, 0]
    rw = _W // params_ref[0, 1]
    row = idx // _W
    col = idx - row * _W
    ok = ((row // rh) == 0) & ((col // rw) == 0)
    val = jnp.where(ok, jnp.float32(1.0), jnp.float32(0.0))
    o_ref[...] = jnp.broadcast_to(val[:, None, None], (_TC_BLK, 1, 128))


def _tc_argmax(size2d, tgt3):
    out3 = pl.pallas_call(
        _tc_body,
        grid=(_TC_PLANES // _TC_BLK,),
        in_specs=[
            pl.BlockSpec(memory_space=pltpu.SMEM),
            pl.BlockSpec((_TC_BLK, _H, _W),
                         lambda g: (g + _SC_PLANES // _TC_BLK, 0, 0)),
        ],
        out_specs=pl.BlockSpec((_TC_BLK, 1, 128), lambda g: (g, 0, 0)),
        out_shape=jax.ShapeDtypeStruct((_TC_PLANES, 1, 128), jnp.float32),
        compiler_params=pltpu.CompilerParams(
            vmem_limit_bytes=100 * 1024 * 1024),
    )(size2d, tgt3)
    return out3[:, 0, 0]


@jax.jit
def kernel(size, target):
    tgt3 = target.reshape(_PLANES, _H, _W)
    parts = []
    if _SC_PLANES:
        s0 = jnp.full((_L,), size[0], jnp.int32)
        s1 = jnp.full((_L,), size[1], jnp.int32)
        sc2d = _sc_argmax(tgt3, s0, s1)
        parts.append(sc2d[:, :_PPW_SC].reshape(_SC_PLANES))
    if _TC_PLANES:
        parts.append(
            _tc_argmax(size.reshape(1, 2), tgt3).reshape(_TC_PLANES))
    return jnp.concatenate(parts).reshape(_B, _C, 1, 1)


# speedup vs baseline: 1.3952x; 1.0252x over previous
import functools

import jax
import jax.numpy as jnp
from jax import lax
from jax.experimental import pallas as pl
from jax.experimental.pallas import tpu as pltpu
from jax.experimental.pallas import tpu_sc as plsc

_B, _C, _H, _W = 2, 96, 512, 512
_PLANES = _B * _C
_PLANE = _H * _W
_NC, _NS, _L = 2, 16, 16
_NW = _NC * _NS
_PPW_SC = 2
_SC_PLANES = _NW * _PPW_SC
_TC_PLANES = _PLANES - _SC_PLANES
_ROWS = 64
_NCHUNK = _H // _ROWS
_ACC = 4
_STRIDE = _ACC * _L
_ITERS = _ROWS * _W // _STRIDE
_UNROLL = 4


def _sc_body(tgt, s0, s1, out, buf0, buf1, s0_v, s1_v, res_v, sem0, sem1):
    cid = lax.axis_index("c")
    sid = lax.axis_index("s")
    wid = sid * _NC + cid
    plane_base = wid * _PPW_SC

    bufs = (buf0, buf1)
    sems = (sem0, sem1)
    lanes = lax.iota(jnp.int32, _L)
    neg_inf = jnp.full((_L,), -jnp.inf, jnp.float32)
    zero_i = jnp.zeros((_L,), jnp.int32)

    pltpu.sync_copy(s0, s0_v)
    pltpu.sync_copy(s1, s1_v)
    ratio_h = lax.div(jnp.full((_L,), _H, jnp.int32), s0_v[...])
    ratio_w = lax.div(jnp.full((_L,), _W, jnp.int32), s1_v[...])

    def start_dma(c):
        j, k = divmod(c, _NCHUNK)
        src = tgt.at[plane_base + j, pl.ds(k * _ROWS, _ROWS), :]
        return pltpu.async_copy(src, bufs[c % 2], sems[c % 2])

    total = _PPW_SC * _NCHUNK
    descs = [None, None]
    descs[0] = start_dma(0)

    res = jnp.zeros((_L,), jnp.float32)

    for j in range(_PPW_SC):
        bv = [neg_inf] * _ACC
        bi = [zero_i] * _ACC
        for k in range(_NCHUNK):
            c = j * _NCHUNK + k
            if c + 1 < total:
                descs[(c + 1) % 2] = start_dma(c + 1)
            descs[c % 2].wait()
            buf = bufs[c % 2]

            def chunk_body(i, carry, buf=buf, k=k):
                v = list(carry[:_ACC])
                ii = list(carry[_ACC:])
                isplat = jnp.full((_L,), i, jnp.int32)
                f0 = (i - k * _ITERS) * _STRIDE
                for a in range(_ACC):
                    f = f0 + a * _L
                    r = lax.shift_right_logical(f, 9)
                    col = lax.bitwise_and(f, 511)
                    x = buf[r, pl.ds(col, _L)]
                    m = x > v[a]
                    v[a] = jnp.where(m, x, v[a])
                    ii[a] = jnp.where(m, isplat, ii[a])
                return tuple(v) + tuple(ii)

            carry = tuple(bv) + tuple(bi)
            carry = plsc.parallel_loop(
                k * _ITERS, (k + 1) * _ITERS, carry=carry,
                unroll=_UNROLL)(chunk_body)
            bv = list(carry[:_ACC])
            bi = list(carry[_ACC:])

        pv = None
        pi = None
        for a in range(_ACC):
            kv = lax.shift_right_logical(bi[a], 9)
            f = (bi[a] & 511) * _STRIDE + a * _L + lanes
            row = kv * _ROWS + lax.shift_right_logical(f, 9)
            col = lax.bitwise_and(f, 511)
            gi = row * _W + col
            if a == 0:
                pv, pi = bv[a], gi
            else:
                take = (bv[a] > pv) | ((bv[a] == pv) & (gi < pi))
                pv = jnp.where(take, bv[a], pv)
                pi = jnp.where(take, gi, pi)

        m = jnp.max(pv, axis=0)
        cand = jnp.where(pv == m, pi, jnp.int32(_PLANE))
        gidx = jnp.min(cand, axis=0)

        gv16 = jnp.full((_L,), gidx, jnp.int32)
        rowv = lax.div(gv16, jnp.full((_L,), _W, jnp.int32))
        colv = gv16 - rowv * _W
        rv = lax.div(rowv, ratio_h)
        cv = lax.div(colv, ratio_w)
        ok = (rv == 0) & (cv == 0)
        val = jnp.where(ok, jnp.float32(1.0), jnp.float32(0.0))
        res = jnp.where(lanes == j, val, res)

    res_v[...] = res
    pltpu.sync_copy(res_v, out.at[wid])


@functools.partial(
    pl.kernel,
    out_type=jax.ShapeDtypeStruct((_NW, _L), jnp.float32),
    mesh=plsc.VectorSubcoreMesh(core_axis_name="c", subcore_axis_name="s"),
    compiler_params=pltpu.CompilerParams(needs_layout_passes=False,
                                         use_tc_tiling_on_sc=True),
    scratch_types=[
        pltpu.VMEM((_ROWS, _W), jnp.float32),
        pltpu.VMEM((_ROWS, _W), jnp.float32),
        pltpu.VMEM((_L,), jnp.int32),
        pltpu.VMEM((_L,), jnp.int32),
        pltpu.VMEM((_L,), jnp.float32),
        pltpu.SemaphoreType.DMA,
        pltpu.SemaphoreType.DMA,
    ],
)
def _sc_argmax(tgt, s0, s1, out, buf0, buf1, s0_v, s1_v, res_v, sem0, sem1):
    _sc_body(tgt, s0, s1, out, buf0, buf1, s0_v, s1_v, res_v, sem0, sem1)


_TC_BLK = 8


def _tc_body(params_ref, x_ref, o_ref):
    x = x_ref[...]
    m = jnp.max(x, axis=(1, 2), keepdims=True)
    iota = (lax.broadcasted_iota(jnp.int32, x.shape, 1) * _W +
            lax.broadcasted_iota(jnp.int32, x.shape, 2))
    idx = jnp.min(jnp.where(x == m, iota, jnp.int32(_PLANE)),
                  axis=(1, 2))
    rh = _H // params_ref[0, 0]
    rw = _W // params_ref[0, 1]
    row = idx // _W
    col = idx - row * _W
    ok = ((row // rh) == 0) & ((col // rw) == 0)
    val = jnp.where(ok, jnp.float32(1.0), jnp.float32(0.0))
    o_ref[...] = jnp.broadcast_to(val[:, None, None], (_TC_BLK, 1, 128))


def _tc_argmax(size2d, tgt3):
    out3 = pl.pallas_call(
        _tc_body,
        grid=(_TC_PLANES // _TC_BLK,),
        in_specs=[
            pl.BlockSpec(memory_space=pltpu.SMEM),
            pl.BlockSpec((_TC_BLK, _H, _W),
                         lambda g: (g + _SC_PLANES // _TC_BLK, 0, 0)),
        ],
        out_specs=pl.BlockSpec((_TC_BLK, 1, 128), lambda g: (g, 0, 0)),
        out_shape=jax.ShapeDtypeStruct((_TC_PLANES, 1, 128), jnp.float32),
        compiler_params=pltpu.CompilerParams(
            vmem_limit_bytes=100 * 1024 * 1024),
    )(size2d, tgt3)
    return out3[:, 0, 0]


@jax.jit
def kernel(size, target):
    tgt3 = target.reshape(_PLANES, _H, _W)
    parts = []
    if _SC_PLANES:
        s0 = jnp.full((_L,), size[0], jnp.int32)
        s1 = jnp.full((_L,), size[1], jnp.int32)
        sc2d = _sc_argmax(tgt3, s0, s1)
        parts.append(sc2d[:, :_PPW_SC].reshape(_SC_PLANES))
    if _TC_PLANES:
        parts.append(
            _tc_argmax(size.reshape(1, 2), tgt3).reshape(_TC_PLANES))
    return jnp.concatenate(parts).reshape(_B, _C, 1, 1)
